# baseline (speedup 1.0000x reference)
import jax
import jax.numpy as jnp
from jax.experimental import pallas as pl
from jax.experimental.pallas import tpu as pltpu

B, S, D = 2, 2048, 1024
H = 256
E = 8
K = 2

TS_GATE = 1024

INTERPRET = False


def _gate_kernel(x_ref, wg_ref, bg_ref, gated_ref, psum_ref):
    x = x_ref[0]
    g = jax.nn.sigmoid(
        jnp.dot(x.astype(jnp.bfloat16), wg_ref[...],
                preferred_element_type=jnp.float32)
        + bg_ref[0]
    )
    gt = x * g
    gated_ref[0] = gt.astype(jnp.bfloat16)
    part = jnp.sum(gt, axis=0, keepdims=True)

    @pl.when(pl.program_id(1) == 0)
    def _():
        psum_ref[0] = part

    @pl.when(pl.program_id(1) != 0)
    def _():
        psum_ref[0] += part


def _router_kernel(ps_ref, w1_ref, b1_ref, w2_ref, b2_ref,
                   gains_ref, probs_ref):
    pooled = ps_ref[:, 0, :] * (1.0 / S)
    h = jnp.tanh(
        jnp.dot(pooled, w1_ref[...], preferred_element_type=jnp.float32)
        + b1_ref[0]
    )
    logits = (jnp.dot(h, w2_ref[...], preferred_element_type=jnp.float32)
              + b2_ref[0])
    m = jnp.max(logits, axis=-1, keepdims=True)
    ex = jnp.exp(logits - m)
    probs = ex / jnp.sum(ex, axis=-1, keepdims=True)
    probs_ref[...] = probs
    eidx = jax.lax.broadcasted_iota(jnp.int32, (B, E), 1)
    v1 = jnp.max(probs, axis=-1, keepdims=True)
    i1 = jnp.min(jnp.where(probs == v1, eidx, E), axis=-1, keepdims=True)
    masked = jnp.where(eidx == i1, -jnp.inf, probs)
    v2 = jnp.max(masked, axis=-1, keepdims=True)
    i2 = jnp.min(jnp.where(masked == v2, eidx, E), axis=-1, keepdims=True)
    wsum = v1 + v2 + 1e-9
    gains_ref[...] = (jnp.where(eidx == i1, v1 / wsum, 0.0)
                      + jnp.where(eidx == i2, v2 / wsum, 0.0))


def _expand_kernel(gains_ref, gated_ref, out_ref):
    b = pl.program_id(0)
    e = pl.program_id(1)
    gt = gated_ref[0].astype(jnp.float32)
    out_ref[0, 0] = gt * gains_ref[b, e]


def kernel(x, Wg, bg, W1, b1, W2, b2):
    wg16 = Wg.astype(jnp.bfloat16)
    bg2 = bg.reshape(1, D)
    b12 = b1.reshape(1, H)
    b22 = b2.reshape(1, E)

    gated, psum = pl.pallas_call(
        _gate_kernel,
        grid=(B, S // TS_GATE),
        in_specs=[
            pl.BlockSpec((1, TS_GATE, D), lambda b, s: (b, s, 0)),
            pl.BlockSpec((D, D), lambda b, s: (0, 0)),
            pl.BlockSpec((1, D), lambda b, s: (0, 0)),
        ],
        out_specs=[
            pl.BlockSpec((1, TS_GATE, D), lambda b, s: (b, s, 0)),
            pl.BlockSpec((1, 1, D), lambda b, s: (b, 0, 0)),
        ],
        out_shape=[
            jax.ShapeDtypeStruct((B, S, D), jnp.bfloat16),
            jax.ShapeDtypeStruct((B, 1, D), jnp.float32),
        ],
        interpret=INTERPRET,
    )(x, wg16, bg2)

    gains, probs = pl.pallas_call(
        _router_kernel,
        in_specs=[
            pl.BlockSpec((B, 1, D), lambda: (0, 0, 0)),
            pl.BlockSpec((D, H), lambda: (0, 0)),
            pl.BlockSpec((1, H), lambda: (0, 0)),
            pl.BlockSpec((H, E), lambda: (0, 0)),
            pl.BlockSpec((1, E), lambda: (0, 0)),
        ],
        out_specs=[
            pl.BlockSpec((B, E), lambda: (0, 0)),
            pl.BlockSpec((B, E), lambda: (0, 0)),
        ],
        out_shape=[
            jax.ShapeDtypeStruct((B, E), jnp.float32),
            jax.ShapeDtypeStruct((B, E), jnp.float32),
        ],
        interpret=INTERPRET,
    )(psum, W1, b12, W2, b22)

    routed = pl.pallas_call(
        _expand_kernel,
        grid=(B, E),
        in_specs=[
            pl.BlockSpec(memory_space=pltpu.SMEM),
            pl.BlockSpec((1, S, D), lambda b, e: (b, 0, 0)),
        ],
        out_specs=pl.BlockSpec((1, 1, S, D), lambda b, e: (e, b, 0, 0)),
        out_shape=jax.ShapeDtypeStruct((E, B, S, D), jnp.float32),
        interpret=INTERPRET,
    )(gains, gated)

    return routed, probs


# speedup vs baseline: 1.0104x; 1.0104x over previous
import jax
import jax.numpy as jnp
from jax.experimental import pallas as pl
from jax.experimental.pallas import tpu as pltpu

B, S, D = 2, 2048, 1024
H = 256
E = 8
K = 2

TS_GATE = 1024
TS_EXP = 256

INTERPRET = False


def _gate_kernel(x_ref, wg_ref, bg_ref, gated_ref, psum_ref):
    x = x_ref[0]
    g = jax.nn.sigmoid(
        jnp.dot(x.astype(jnp.bfloat16), wg_ref[...],
                preferred_element_type=jnp.float32)
        + bg_ref[0]
    )
    gt = x * g
    gated_ref[0] = gt.astype(jnp.bfloat16)
    part = jnp.sum(gt, axis=0, keepdims=True)

    @pl.when(pl.program_id(1) == 0)
    def _():
        psum_ref[0] = part

    @pl.when(pl.program_id(1) != 0)
    def _():
        psum_ref[0] += part


def _router_kernel(ps_ref, w1_ref, b1_ref, w2_ref, b2_ref,
                   gains_ref, probs_ref):
    pooled = ps_ref[:, 0, :] * (1.0 / S)
    h = jnp.tanh(
        jnp.dot(pooled, w1_ref[...], preferred_element_type=jnp.float32)
        + b1_ref[0]
    )
    logits = (jnp.dot(h, w2_ref[...], preferred_element_type=jnp.float32)
              + b2_ref[0])
    m = jnp.max(logits, axis=-1, keepdims=True)
    ex = jnp.exp(logits - m)
    probs = ex / jnp.sum(ex, axis=-1, keepdims=True)
    probs_ref[...] = probs
    eidx = jax.lax.broadcasted_iota(jnp.int32, (B, E), 1)
    v1 = jnp.max(probs, axis=-1, keepdims=True)
    i1 = jnp.min(jnp.where(probs == v1, eidx, E), axis=-1, keepdims=True)
    masked = jnp.where(eidx == i1, -jnp.inf, probs)
    v2 = jnp.max(masked, axis=-1, keepdims=True)
    i2 = jnp.min(jnp.where(masked == v2, eidx, E), axis=-1, keepdims=True)
    wsum = v1 + v2 + 1e-9
    gains_ref[...] = (jnp.where(eidx == i1, v1 / wsum, 0.0)
                      + jnp.where(eidx == i2, v2 / wsum, 0.0))


def _expand_kernel(gains_ref, gated_ref, out_ref):
    b = pl.program_id(0)
    gt = gated_ref[0].astype(jnp.float32)
    for e in range(E):
        out_ref[e, 0] = gt * gains_ref[b, e]


def kernel(x, Wg, bg, W1, b1, W2, b2):
    wg16 = Wg.astype(jnp.bfloat16)
    bg2 = bg.reshape(1, D)
    b12 = b1.reshape(1, H)
    b22 = b2.reshape(1, E)

    gated, psum = pl.pallas_call(
        _gate_kernel,
        grid=(B, S // TS_GATE),
        in_specs=[
            pl.BlockSpec((1, TS_GATE, D), lambda b, s: (b, s, 0)),
            pl.BlockSpec((D, D), lambda b, s: (0, 0)),
            pl.BlockSpec((1, D), lambda b, s: (0, 0)),
        ],
        out_specs=[
            pl.BlockSpec((1, TS_GATE, D), lambda b, s: (b, s, 0)),
            pl.BlockSpec((1, 1, D), lambda b, s: (b, 0, 0)),
        ],
        out_shape=[
            jax.ShapeDtypeStruct((B, S, D), jnp.bfloat16),
            jax.ShapeDtypeStruct((B, 1, D), jnp.float32),
        ],
        interpret=INTERPRET,
    )(x, wg16, bg2)

    gains, probs = pl.pallas_call(
        _router_kernel,
        in_specs=[
            pl.BlockSpec((B, 1, D), lambda: (0, 0, 0)),
            pl.BlockSpec((D, H), lambda: (0, 0)),
            pl.BlockSpec((1, H), lambda: (0, 0)),
            pl.BlockSpec((H, E), lambda: (0, 0)),
            pl.BlockSpec((1, E), lambda: (0, 0)),
        ],
        out_specs=[
            pl.BlockSpec((B, E), lambda: (0, 0)),
            pl.BlockSpec((B, E), lambda: (0, 0)),
        ],
        out_shape=[
            jax.ShapeDtypeStruct((B, E), jnp.float32),
            jax.ShapeDtypeStruct((B, E), jnp.float32),
        ],
        interpret=INTERPRET,
    )(psum, W1, b12, W2, b22)

    routed = pl.pallas_call(
        _expand_kernel,
        grid=(B, S // TS_EXP),
        in_specs=[
            pl.BlockSpec(memory_space=pltpu.SMEM),
            pl.BlockSpec((1, TS_EXP, D), lambda b, s: (b, s, 0)),
        ],
        out_specs=pl.BlockSpec((E, 1, TS_EXP, D), lambda b, s: (0, b, s, 0)),
        out_shape=jax.ShapeDtypeStruct((E, B, S, D), jnp.float32),
        interpret=INTERPRET,
    )(gains, gated)

    return routed, probs


# speedup vs baseline: 1.0609x; 1.0500x over previous
import jax
import jax.numpy as jnp
from jax.experimental import pallas as pl
from jax.experimental.pallas import tpu as pltpu

B, S, D = 2, 2048, 1024
H = 256
E = 8
K = 2

TS_GATE = 1024
TS_EXP = 256

INTERPRET = False


def _gate_kernel(x_ref, wg_ref, bg_ref, gated_ref, psum_ref):
    x = x_ref[0]
    g = jax.nn.sigmoid(
        jnp.dot(x.astype(jnp.bfloat16), wg_ref[...].astype(jnp.bfloat16),
                preferred_element_type=jnp.float32)
        + bg_ref[0]
    )
    gt = x * g
    gated_ref[0] = gt.astype(jnp.bfloat16)
    part = jnp.sum(gt, axis=0, keepdims=True)

    @pl.when(pl.program_id(1) == 0)
    def _():
        psum_ref[0] = part

    @pl.when(pl.program_id(1) != 0)
    def _():
        psum_ref[0] += part


def _router_kernel(ps_ref, w1_ref, b1_ref, w2_ref, b2_ref,
                   gains_ref, probs_ref):
    pooled = ps_ref[:, 0, :] * (1.0 / S)
    h = jnp.tanh(
        jnp.dot(pooled, w1_ref[...], preferred_element_type=jnp.float32)
        + b1_ref[0]
    )
    logits = (jnp.dot(h, w2_ref[...], preferred_element_type=jnp.float32)
              + b2_ref[0])
    m = jnp.max(logits, axis=-1, keepdims=True)
    ex = jnp.exp(logits - m)
    probs = ex / jnp.sum(ex, axis=-1, keepdims=True)
    probs_ref[...] = probs
    eidx = jax.lax.broadcasted_iota(jnp.int32, (B, E), 1)
    v1 = jnp.max(probs, axis=-1, keepdims=True)
    i1 = jnp.min(jnp.where(probs == v1, eidx, E), axis=-1, keepdims=True)
    masked = jnp.where(eidx == i1, -jnp.inf, probs)
    v2 = jnp.max(masked, axis=-1, keepdims=True)
    i2 = jnp.min(jnp.where(masked == v2, eidx, E), axis=-1, keepdims=True)
    wsum = v1 + v2 + 1e-9
    gains_ref[...] = (jnp.where(eidx == i1, v1 / wsum, 0.0)
                      + jnp.where(eidx == i2, v2 / wsum, 0.0))


def _expand_kernel(gains_ref, gated_ref, out_ref):
    b = pl.program_id(0)
    gt = gated_ref[0].astype(jnp.float32)
    for e in range(E):
        out_ref[e, 0] = gt * gains_ref[b, e]


def kernel(x, Wg, bg, W1, b1, W2, b2):
    bg2 = bg.reshape(1, D)
    b12 = b1.reshape(1, H)
    b22 = b2.reshape(1, E)

    gated, psum = pl.pallas_call(
        _gate_kernel,
        grid=(B, S // TS_GATE),
        in_specs=[
            pl.BlockSpec((1, TS_GATE, D), lambda b, s: (b, s, 0)),
            pl.BlockSpec((D, D), lambda b, s: (0, 0)),
            pl.BlockSpec((1, D), lambda b, s: (0, 0)),
        ],
        out_specs=[
            pl.BlockSpec((1, TS_GATE, D), lambda b, s: (b, s, 0)),
            pl.BlockSpec((1, 1, D), lambda b, s: (b, 0, 0)),
        ],
        out_shape=[
            jax.ShapeDtypeStruct((B, S, D), jnp.bfloat16),
            jax.ShapeDtypeStruct((B, 1, D), jnp.float32),
        ],
        interpret=INTERPRET,
    )(x, Wg, bg2)

    gains, probs = pl.pallas_call(
        _router_kernel,
        in_specs=[
            pl.BlockSpec((B, 1, D), lambda: (0, 0, 0)),
            pl.BlockSpec((D, H), lambda: (0, 0)),
            pl.BlockSpec((1, H), lambda: (0, 0)),
            pl.BlockSpec((H, E), lambda: (0, 0)),
            pl.BlockSpec((1, E), lambda: (0, 0)),
        ],
        out_specs=[
            pl.BlockSpec((B, E), lambda: (0, 0)),
            pl.BlockSpec((B, E), lambda: (0, 0)),
        ],
        out_shape=[
            jax.ShapeDtypeStruct((B, E), jnp.float32),
            jax.ShapeDtypeStruct((B, E), jnp.float32),
        ],
        interpret=INTERPRET,
    )(psum, W1, b12, W2, b22)

    routed = pl.pallas_call(
        _expand_kernel,
        grid=(B, S // TS_EXP),
        in_specs=[
            pl.BlockSpec(memory_space=pltpu.SMEM),
            pl.BlockSpec((1, TS_EXP, D), lambda b, s: (b, s, 0)),
        ],
        out_specs=pl.BlockSpec((E, 1, TS_EXP, D), lambda b, s: (0, b, s, 0)),
        out_shape=jax.ShapeDtypeStruct((E, B, S, D), jnp.float32),
        interpret=INTERPRET,
    )(gains, gated)

    return routed, probs
